# baseline (speedup 1.0000x reference)
import functools

import jax
import jax.numpy as jnp
from jax import lax
from jax.experimental import pallas as pl
from jax.experimental.pallas import tpu as pltpu
from jax.experimental.pallas import tpu_sc as plsc

B = 1024
C = 100000
_F = 1.0 / (1.0 + max(5.0, 1500.0 / 1.1))

_NC = 2
_NS = 16
_L = 16
_NW = _NC * _NS
_BPW = B // _NW


@functools.cache
def _build_sc_gather():
    mesh = plsc.VectorSubcoreMesh(core_axis_name="c", subcore_axis_name="s")

    @functools.partial(
        pl.kernel,
        mesh=mesh,
        out_type=(
            jax.ShapeDtypeStruct((B,), jnp.float32),
            jax.ShapeDtypeStruct((B,), jnp.float32),
        ),
        scratch_types=[
            pltpu.VMEM((_BPW,), jnp.int32),
            pltpu.VMEM((_L, 8, 128), jnp.float32),
            pltpu.VMEM((_L, 8, 128), jnp.float32),
            pltpu.VMEM((_BPW,), jnp.float32),
            pltpu.VMEM((_BPW,), jnp.float32),
            pltpu.SemaphoreType.DMA,
            pltpu.SemaphoreType.DMA,
        ],
        compiler_params=pltpu.CompilerParams(use_tc_tiling_on_sc=True,
                                             needs_layout_passes=False),
    )
    def sc_gather(tgt_hbm, cost_hbm, psit_hbm, cos_out, psi_out,
                  tgt_v, tile_c, tile_p, ct_v, pt_v, sem_c, sem_p):
        wid = lax.axis_index("s") * _NC + lax.axis_index("c")
        base = wid * _BPW
        col0 = pl.multiple_of((base // 128) * 128, 128)
        pltpu.sync_copy(tgt_hbm.at[pl.ds(base, _BPW)], tgt_v)
        lanes = lax.iota(jnp.int32, _L)
        for g in range(_BPW // _L):
            t16 = tgt_v[pl.ds(g * _L, _L)]
            r016 = (t16 >> 3) << 3
            copies = []
            for k in range(_L):
                r0 = pl.multiple_of(r016[k], 8)
                copies.append(pltpu.async_copy(
                    cost_hbm.at[pl.ds(r0, 8), pl.ds(col0, 128)],
                    tile_c.at[k], sem_c))
                copies.append(pltpu.async_copy(
                    psit_hbm.at[pl.ds(r0, 8), pl.ds(col0, 128)],
                    tile_p.at[k], sem_p))
            for cp in copies:
                cp.wait()
            sub16 = t16 & 7
            off16 = lanes + (base % 128 + g * _L)
            ct_v[pl.ds(g * _L, _L)] = plsc.load_gather(
                tile_c, [lanes, sub16, off16])
            pt_v[pl.ds(g * _L, _L)] = plsc.load_gather(
                tile_p, [lanes, sub16, off16])
        pltpu.sync_copy(ct_v, cos_out.at[pl.ds(base, _BPW)])
        pltpu.sync_copy(pt_v, psi_out.at[pl.ds(base, _BPW)])

    return sc_gather


_CB = 1000
_NJ = C // _CB


def _tc_body(cos_t_ref, psi_t_ref, x_ref, out_ref, acc_ref):
    j = pl.program_id(0)

    @pl.when(j == 0)
    def _init():
        acc_ref[...] = jnp.zeros_like(acc_ref)

    e = jnp.exp(x_ref[...])
    acc_ref[...] += jnp.sum(e.reshape(_CB // 8, 8, B), axis=0)

    @pl.when(j == _NJ - 1)
    def _finish():
        s = jnp.sum(acc_ref[...], axis=0, keepdims=True)
        ct = cos_t_ref[...]
        pt = psi_t_ref[...]
        v = ct + _F * (pt - ct)
        strue = s - jnp.exp(ct) + jnp.exp(v)
        logpt = v - jnp.log(strue)
        out_ref[...] = jnp.reshape(-jnp.sum(logpt) * (1.0 / B), (1, 1))


def kernel(cos_theta, psi_theta, target):
    tgt = target.reshape(-1).astype(jnp.int32)
    cos_tr = jnp.swapaxes(cos_theta, 0, 1)
    psi_tr = jnp.swapaxes(psi_theta, 0, 1)
    ct, pt = _build_sc_gather()(tgt, cos_tr, psi_tr)
    out = pl.pallas_call(
        _tc_body,
        grid=(_NJ,),
        in_specs=[
            pl.BlockSpec((1, B), lambda j: (0, 0)),
            pl.BlockSpec((1, B), lambda j: (0, 0)),
            pl.BlockSpec((_CB, B), lambda j: (j, 0)),
        ],
        out_specs=pl.BlockSpec((1, 1), lambda j: (0, 0)),
        out_shape=jax.ShapeDtypeStruct((1, 1), jnp.float32),
        scratch_shapes=[pltpu.VMEM((8, B), jnp.float32)],
    )(ct.reshape(1, B), pt.reshape(1, B), cos_tr)
    return out[0, 0]


# speedup vs baseline: 1.0918x; 1.0918x over previous
import functools

import jax
import jax.numpy as jnp
from jax import lax
from jax.experimental import pallas as pl
from jax.experimental.pallas import tpu as pltpu
from jax.experimental.pallas import tpu_sc as plsc

B = 1024
C = 100000
_F = 1.0 / (1.0 + max(5.0, 1500.0 / 1.1))

_NC = 2
_NS = 16
_L = 16
_NW = _NC * _NS
_BPW = B // _NW


@functools.cache
def _build_sc_gather():
    mesh = plsc.VectorSubcoreMesh(core_axis_name="c", subcore_axis_name="s")

    @functools.partial(
        pl.kernel,
        mesh=mesh,
        out_type=(
            jax.ShapeDtypeStruct((B,), jnp.float32),
            jax.ShapeDtypeStruct((B,), jnp.float32),
        ),
        scratch_types=[
            pltpu.VMEM((_BPW,), jnp.int32),
            pltpu.VMEM((_L, 8, 128), jnp.float32),
            pltpu.VMEM((_L, 8, 128), jnp.float32),
            pltpu.VMEM((_BPW,), jnp.float32),
            pltpu.VMEM((_BPW,), jnp.float32),
            pltpu.SemaphoreType.DMA,
            pltpu.SemaphoreType.DMA,
        ],
        compiler_params=pltpu.CompilerParams(use_tc_tiling_on_sc=True,
                                             needs_layout_passes=False),
    )
    def sc_gather(tgt_hbm, cost_hbm, psit_hbm, cos_out, psi_out,
                  tgt_v, tile_c, tile_p, ct_v, pt_v, sem_c, sem_p):
        wid = lax.axis_index("s") * _NC + lax.axis_index("c")
        base = wid * _BPW
        col0 = pl.multiple_of((base // 128) * 128, 128)
        pltpu.sync_copy(tgt_hbm.at[pl.ds(base, _BPW)], tgt_v)
        lanes = lax.iota(jnp.int32, _L)
        for g in range(_BPW // _L):
            t16 = tgt_v[pl.ds(g * _L, _L)]
            r016 = (t16 >> 3) << 3
            copies = []
            for k in range(_L):
                r0 = pl.multiple_of(r016[k], 8)
                copies.append(pltpu.async_copy(
                    cost_hbm.at[pl.ds(r0, 8), pl.ds(col0, 128)],
                    tile_c.at[k], sem_c))
                copies.append(pltpu.async_copy(
                    psit_hbm.at[pl.ds(r0, 8), pl.ds(col0, 128)],
                    tile_p.at[k], sem_p))
            for cp in copies:
                cp.wait()
            sub16 = t16 & 7
            off16 = lanes + (base % 128 + g * _L)
            ct_v[pl.ds(g * _L, _L)] = plsc.load_gather(
                tile_c, [lanes, sub16, off16])
            pt_v[pl.ds(g * _L, _L)] = plsc.load_gather(
                tile_p, [lanes, sub16, off16])
        pltpu.sync_copy(ct_v, cos_out.at[pl.ds(base, _BPW)])
        pltpu.sync_copy(pt_v, psi_out.at[pl.ds(base, _BPW)])

    return sc_gather


_CB = 5000
_NJ = C // _CB


def _tc_body(cos_t_ref, psi_t_ref, x_ref, out_ref, acc_ref):
    j = pl.program_id(0)

    @pl.when(j == 0)
    def _init():
        acc_ref[...] = jnp.zeros_like(acc_ref)

    e = jnp.exp(x_ref[...])
    acc_ref[...] += jnp.sum(e.reshape(_CB // 8, 8, B), axis=0)

    @pl.when(j == _NJ - 1)
    def _finish():
        s = jnp.sum(acc_ref[...], axis=0, keepdims=True)
        ct = cos_t_ref[...]
        pt = psi_t_ref[...]
        v = ct + _F * (pt - ct)
        strue = s - jnp.exp(ct) + jnp.exp(v)
        logpt = v - jnp.log(strue)
        out_ref[...] = jnp.reshape(-jnp.sum(logpt) * (1.0 / B), (1, 1))


def kernel(cos_theta, psi_theta, target):
    tgt = target.reshape(-1).astype(jnp.int32)
    cos_tr = jnp.swapaxes(cos_theta, 0, 1)
    psi_tr = jnp.swapaxes(psi_theta, 0, 1)
    ct, pt = _build_sc_gather()(tgt, cos_tr, psi_tr)
    out = pl.pallas_call(
        _tc_body,
        grid=(_NJ,),
        in_specs=[
            pl.BlockSpec((1, B), lambda j: (0, 0)),
            pl.BlockSpec((1, B), lambda j: (0, 0)),
            pl.BlockSpec((_CB, B), lambda j: (j, 0)),
        ],
        out_specs=pl.BlockSpec((1, 1), lambda j: (0, 0)),
        out_shape=jax.ShapeDtypeStruct((1, 1), jnp.float32),
        scratch_shapes=[pltpu.VMEM((8, B), jnp.float32)],
    )(ct.reshape(1, B), pt.reshape(1, B), cos_tr)
    return out[0, 0]


# speedup vs baseline: 1.2230x; 1.1201x over previous
import functools

import jax
import jax.numpy as jnp
from jax import lax
from jax.experimental import pallas as pl
from jax.experimental.pallas import tpu as pltpu
from jax.experimental.pallas import tpu_sc as plsc

B = 1024
C = 100000
_F = 1.0 / (1.0 + max(5.0, 1500.0 / 1.1))

_NC = 2
_NS = 16
_L = 16
_NW = _NC * _NS
_BPW = B // _NW


@functools.cache
def _build_sc_gather():
    mesh = plsc.VectorSubcoreMesh(core_axis_name="c", subcore_axis_name="s")

    @functools.partial(
        pl.kernel,
        mesh=mesh,
        out_type=(
            jax.ShapeDtypeStruct((B,), jnp.float32),
            jax.ShapeDtypeStruct((B,), jnp.float32),
        ),
        scratch_types=[
            pltpu.VMEM((_BPW,), jnp.int32),
            pltpu.VMEM((_L, 8, 128), jnp.float32),
            pltpu.VMEM((_L, 8, 128), jnp.float32),
            pltpu.VMEM((_BPW,), jnp.float32),
            pltpu.VMEM((_BPW,), jnp.float32),
            pltpu.SemaphoreType.DMA,
            pltpu.SemaphoreType.DMA,
        ],
        compiler_params=pltpu.CompilerParams(use_tc_tiling_on_sc=True,
                                             needs_layout_passes=False),
    )
    def sc_gather(tgt_hbm, cost_hbm, psit_hbm, cos_out, psi_out,
                  tgt_v, tile_c, tile_p, ct_v, pt_v, sem_c, sem_p):
        wid = lax.axis_index("s") * _NC + lax.axis_index("c")
        base = wid * _BPW
        col0 = pl.multiple_of((base // 128) * 128, 128)
        pltpu.sync_copy(tgt_hbm.at[pl.ds(base, _BPW)], tgt_v)
        lanes = lax.iota(jnp.int32, _L)
        for g in range(_BPW // _L):
            t16 = tgt_v[pl.ds(g * _L, _L)]
            r016 = (t16 >> 3) << 3
            copies = []
            for k in range(_L):
                r0 = pl.multiple_of(r016[k], 8)
                copies.append(pltpu.async_copy(
                    cost_hbm.at[pl.ds(r0, 8), pl.ds(col0, 128)],
                    tile_c.at[k], sem_c))
                copies.append(pltpu.async_copy(
                    psit_hbm.at[pl.ds(r0, 8), pl.ds(col0, 128)],
                    tile_p.at[k], sem_p))
            for cp in copies:
                cp.wait()
            sub16 = t16 & 7
            off16 = lanes + (base % 128 + g * _L)
            ct_v[pl.ds(g * _L, _L)] = plsc.load_gather(
                tile_c, [lanes, sub16, off16])
            pt_v[pl.ds(g * _L, _L)] = plsc.load_gather(
                tile_p, [lanes, sub16, off16])
        pltpu.sync_copy(ct_v, cos_out.at[pl.ds(base, _BPW)])
        pltpu.sync_copy(pt_v, psi_out.at[pl.ds(base, _BPW)])

    return sc_gather


_CB = 2000
_NJ = C // _CB


def _tc_body(x_ref, out_ref, acc_ref):
    j = pl.program_id(0)

    @pl.when(j == 0)
    def _init():
        acc_ref[...] = jnp.zeros_like(acc_ref)

    e = jnp.exp(x_ref[...])
    acc_ref[...] += jnp.sum(e.reshape(_CB // 8, 8, B), axis=0)

    @pl.when(j == _NJ - 1)
    def _finish():
        out_ref[...] = jnp.sum(acc_ref[...], axis=0, keepdims=True)


def _combine_body(s_ref, cos_t_ref, psi_t_ref, out_ref):
    s = s_ref[...]
    ct = cos_t_ref[...]
    pt = psi_t_ref[...]
    v = ct + _F * (pt - ct)
    strue = s - jnp.exp(ct) + jnp.exp(v)
    logpt = v - jnp.log(strue)
    out_ref[...] = jnp.reshape(-jnp.sum(logpt) * (1.0 / B), (1, 1))


def kernel(cos_theta, psi_theta, target):
    tgt = target.reshape(-1).astype(jnp.int32)
    cos_tr = jnp.swapaxes(cos_theta, 0, 1)
    psi_tr = jnp.swapaxes(psi_theta, 0, 1)
    ct, pt = _build_sc_gather()(tgt, cos_tr, psi_tr)
    s = pl.pallas_call(
        _tc_body,
        grid=(_NJ,),
        in_specs=[pl.BlockSpec((_CB, B), lambda j: (j, 0))],
        out_specs=pl.BlockSpec((1, B), lambda j: (0, 0)),
        out_shape=jax.ShapeDtypeStruct((1, B), jnp.float32),
        scratch_shapes=[pltpu.VMEM((8, B), jnp.float32)],
    )(cos_tr)
    out = pl.pallas_call(
        _combine_body,
        out_shape=jax.ShapeDtypeStruct((1, 1), jnp.float32),
    )(s, ct.reshape(1, B), pt.reshape(1, B))
    return out[0, 0]
